# baseline (speedup 1.0000x reference)
import functools

import jax
import jax.numpy as jnp
from jax import lax
from jax.experimental import pallas as pl
from jax.experimental.pallas import tpu as pltpu
from jax.experimental.pallas import tpu_sc as plsc

_PC_RANGE = [-51.2, -51.2, -5.0, 51.2, 51.2, 3.0]
_IM_H, _IM_W = 512, 1408
_EPS = 1e-05

_NQ = 40000
_NW = 32
_CH = 1280
_NQP = _NW * _CH
_NVEC = _CH // 16
_NP = 4
_NC = 6
_NROW = _NP * _NC


def _tec_body(xyz_hbm, mats_hbm, u_hbm, v_hbm, mask_hbm, cnt_hbm,
              xyz_v, mats_v, u_v, v_v, mask_v, cnt_v, in_sem):
    wid = lax.axis_index("s") * 2 + lax.axis_index("c")
    base = pl.multiple_of(wid * _CH, 128)

    handles = [pltpu.async_copy(
        xyz_hbm.at[pl.ds(r * _NQP + base, _CH)], xyz_v.at[r], in_sem)
        for r in range(12)]
    handles.append(pltpu.async_copy(mats_hbm, mats_v, in_sem))
    for h in handles:
        h.wait()

    def bf16_rne(x):
        bits = plsc.bitcast(x, jnp.int32)
        rnd = bits + jnp.int32(0x7FFF) + (jnp.right_shift(bits, 16) & 1)
        rnd = rnd & jnp.int32(-65536)
        return plsc.bitcast(rnd, jnp.float32)

    def fast_rcp(d):
        bits = plsc.bitcast(d, jnp.int32)
        r = plsc.bitcast(jnp.int32(0x7EF311C3) - bits, jnp.float32)
        r = r * (2.0 - d * r)
        r = r * (2.0 - d * r)
        return r

    _SCALES = (_PC_RANGE[3] - _PC_RANGE[0],
               _PC_RANGE[4] - _PC_RANGE[1],
               _PC_RANGE[5] - _PC_RANGE[2])
    _SHIFTS = (_PC_RANGE[0], _PC_RANGE[1], _PC_RANGE[2])

    @plsc.parallel_loop(0, _CH, step=16)
    def pre(q0):
        for comp in range(3):
            for dd in range(_NP):
                r = comp * _NP + dd
                w = xyz_v[r, pl.ds(q0, 16)] * _SCALES[comp] + _SHIFTS[comp]
                xyz_v[r, pl.ds(q0, 16)] = bf16_rne(w)

    inv_w = float(1.0 / _IM_W)
    inv_h = float(1.0 / _IM_H)

    for g in range(1):
        cams = (0, 1, 2, 3, 4, 5)
        coef = [[mats_v[pl.ds(((c * 3) * 4 + k) * 16, 16)] for k in range(12)]
                for c in cams]

        @plsc.parallel_loop(0, _CH, step=16, unroll=2)
        def body(q0, g=g, cams=cams, coef=coef):
            s = jnp.zeros((16,), jnp.float32)
            for d in range(_NP):
                wx = xyz_v[0 * _NP + d, pl.ds(q0, 16)]
                wy = xyz_v[1 * _NP + d, pl.ds(q0, 16)]
                wz = xyz_v[2 * _NP + d, pl.ds(q0, 16)]
                for cc in range(len(cams)):
                    A = coef[cc]
                    px = (((A[0] * wx) + (A[1] * wy)) + (A[2] * wz)) + A[3]
                    py = (((A[4] * wx) + (A[5] * wy)) + (A[6] * wz)) + A[7]
                    pz = (((A[8] * wx) + (A[9] * wy)) + (A[10] * wz)) + A[11]
                    denom = jnp.maximum(pz, _EPS)
                    rec = fast_rcp(denom)
                    u = (px * rec) * inv_w
                    v = (py * rec) * inv_h
                    m = ((pz > _EPS)
                         & (v > 0.0) & (v < 1.0)
                         & (u < 1.0) & (u > 0.0))
                    mf = jnp.where(m, 1.0, 0.0).astype(jnp.float32)
                    s = s + mf
                    row = d * _NC + cams[cc]
                    u_v[row, pl.ds(q0, 16)] = jnp.clip(u, -2.1, 2.1)
                    v_v[row, pl.ds(q0, 16)] = jnp.clip(v, -2.1, 2.1)
                    mask_v[row, pl.ds(q0, 16)] = mf
            cnt_v[pl.ds(q0, 16)] = s

    pltpu.sync_copy(u_v, u_hbm.at[:, pl.ds(base, _CH)])
    pltpu.sync_copy(v_v, v_hbm.at[:, pl.ds(base, _CH)])
    pltpu.sync_copy(mask_v, mask_hbm.at[:, pl.ds(base, _CH)])
    pltpu.sync_copy(cnt_v, cnt_hbm.at[pl.ds(base, _CH)])


@functools.cache
def _get_sc_call():
    return pl.kernel(
        _tec_body,
        out_type=(
            jax.ShapeDtypeStruct((_NROW, _NQP), jnp.float32),
            jax.ShapeDtypeStruct((_NROW, _NQP), jnp.float32),
            jax.ShapeDtypeStruct((_NROW, _NQP), jnp.float32),
            jax.ShapeDtypeStruct((_NQP,), jnp.float32),
        ),
        mesh=plsc.VectorSubcoreMesh(core_axis_name="c", subcore_axis_name="s",
                                    num_cores=2, num_subcores=16),
        compiler_params=pltpu.CompilerParams(needs_layout_passes=False),
        scratch_types=[
            pltpu.VMEM((12, _CH), jnp.float32),
            pltpu.VMEM((72 * 16,), jnp.float32),
            pltpu.VMEM((_NROW, _CH), jnp.float32),
            pltpu.VMEM((_NROW, _CH), jnp.float32),
            pltpu.VMEM((_NROW, _CH), jnp.float32),
            pltpu.VMEM((_CH,), jnp.float32),
            pltpu.SemaphoreType.DMA,
        ],
    )


@jax.jit
def kernel(reference_points, ego2img):
    rp = reference_points.astype(jnp.float32)[0]
    xyz = jnp.transpose(rp, (2, 0, 1)).reshape(12, _NQ)
    xyz = jnp.pad(xyz, ((0, 0), (0, _NQP - _NQ))).reshape(-1)

    m = ego2img.astype(jnp.float32)[0, :, :3, :]
    bits = m.view(jnp.int32)
    bits = (bits + jnp.int32(0x7FFF) + (jnp.right_shift(bits, 16) & 1))
    bits = bits & jnp.int32(-65536)
    mq = bits.view(jnp.float32)
    mats = jnp.broadcast_to(mq.reshape(72, 1), (72, 16)).reshape(-1)

    u_p, v_p, mask_f, cnt = _get_sc_call()(xyz, mats)

    u4 = u_p[:, :_NQ].reshape(_NP, 1, _NC, _NQ, 1)
    v4 = v_p[:, :_NQ].reshape(_NP, 1, _NC, _NQ, 1)
    parity = lax.broadcasted_iota(jnp.int32, (_NP, 1, _NC, _NQ, 2), 4)
    rpc = jnp.where(parity == 0, u4, v4)
    bev_mask = mask_f[:, :_NQ].reshape(_NP, 1, _NC, _NQ, 1).astype(bool)
    counts = cnt[:_NQ].reshape(1, _NQ)
    return rpc, bev_mask, counts


# speedup vs baseline: 1.0101x; 1.0101x over previous
import functools

import jax
import jax.numpy as jnp
from jax import lax
from jax.experimental import pallas as pl
from jax.experimental.pallas import tpu as pltpu
from jax.experimental.pallas import tpu_sc as plsc

_PC_RANGE = [-51.2, -51.2, -5.0, 51.2, 51.2, 3.0]
_IM_H, _IM_W = 512, 1408
_EPS = 1e-05

_NQ = 40000
_NW = 32
_CH = 1280
_NQP = _NW * _CH
_NVEC = _CH // 16
_NP = 4
_NC = 6
_NROW = _NP * _NC


def _tec_body(xyz_hbm, mats_hbm, u_hbm, v_hbm, mask_hbm, cnt_hbm,
              xyz_v, mats_v, u_v, v_v, mask_v, cnt_v, in_sem):
    wid = lax.axis_index("s") * 2 + lax.axis_index("c")
    base = pl.multiple_of(wid * _CH, 128)

    handles = [pltpu.async_copy(
        xyz_hbm.at[pl.ds(r * _NQP + base, _CH)], xyz_v.at[r], in_sem)
        for r in range(12)]
    handles.append(pltpu.async_copy(mats_hbm, mats_v, in_sem))
    for h in handles:
        h.wait()

    def bf16_rne(x):
        bits = plsc.bitcast(x, jnp.int32)
        rnd = bits + jnp.int32(0x7FFF) + (jnp.right_shift(bits, 16) & 1)
        rnd = rnd & jnp.int32(-65536)
        return plsc.bitcast(rnd, jnp.float32)

    def fast_rcp(d):
        bits = plsc.bitcast(d, jnp.int32)
        r = plsc.bitcast(jnp.int32(0x7EF311C3) - bits, jnp.float32)
        r = r * (2.0 - d * r)
        r = r * (2.0 - d * r)
        return r

    _SCALES = (_PC_RANGE[3] - _PC_RANGE[0],
               _PC_RANGE[4] - _PC_RANGE[1],
               _PC_RANGE[5] - _PC_RANGE[2])
    _SHIFTS = (_PC_RANGE[0], _PC_RANGE[1], _PC_RANGE[2])

    @plsc.parallel_loop(0, _CH, step=16)
    def pre(q0):
        for comp in range(3):
            for dd in range(_NP):
                r = comp * _NP + dd
                w = xyz_v[r, pl.ds(q0, 16)] * _SCALES[comp] + _SHIFTS[comp]
                xyz_v[r, pl.ds(q0, 16)] = bf16_rne(w)

    inv_w = float(1.0 / _IM_W)
    inv_h = float(1.0 / _IM_H)

    for g in range(2):
        cams = (3 * g, 3 * g + 1, 3 * g + 2)
        coef = [[mats_v[pl.ds(((c * 3) * 4 + k) * 16, 16)] for k in range(12)]
                for c in cams]

        @plsc.parallel_loop(0, _CH, step=16, unroll=2)
        def body(q0, g=g, cams=cams, coef=coef):
            s = jnp.zeros((16,), jnp.float32)
            for d in range(_NP):
                wx = xyz_v[0 * _NP + d, pl.ds(q0, 16)]
                wy = xyz_v[1 * _NP + d, pl.ds(q0, 16)]
                wz = xyz_v[2 * _NP + d, pl.ds(q0, 16)]
                for cc in range(len(cams)):
                    A = coef[cc]
                    px = (((A[0] * wx) + (A[1] * wy)) + (A[2] * wz)) + A[3]
                    py = (((A[4] * wx) + (A[5] * wy)) + (A[6] * wz)) + A[7]
                    pz = (((A[8] * wx) + (A[9] * wy)) + (A[10] * wz)) + A[11]
                    denom = jnp.maximum(pz, _EPS)
                    rec = fast_rcp(denom)
                    u = (px * rec) * inv_w
                    v = (py * rec) * inv_h
                    m = ((pz > _EPS)
                         & (v > 0.0) & (v < 1.0)
                         & (u < 1.0) & (u > 0.0))
                    mf = jnp.where(m, 1.0, 0.0).astype(jnp.float32)
                    s = s + mf
                    row = d * _NC + cams[cc]
                    u_v[row, pl.ds(q0, 16)] = jnp.clip(u, -2.1, 2.1)
                    v_v[row, pl.ds(q0, 16)] = jnp.clip(v, -2.1, 2.1)
                    mask_v[row, pl.ds(q0, 16)] = mf
            if g == 0:
                cnt_v[pl.ds(q0, 16)] = s
            else:
                cnt_v[pl.ds(q0, 16)] = cnt_v[pl.ds(q0, 16)] + s

    pltpu.sync_copy(u_v, u_hbm.at[:, pl.ds(base, _CH)])
    pltpu.sync_copy(v_v, v_hbm.at[:, pl.ds(base, _CH)])
    pltpu.sync_copy(mask_v, mask_hbm.at[:, pl.ds(base, _CH)])
    pltpu.sync_copy(cnt_v, cnt_hbm.at[pl.ds(base, _CH)])


@functools.cache
def _get_sc_call():
    return pl.kernel(
        _tec_body,
        out_type=(
            jax.ShapeDtypeStruct((_NROW, _NQP), jnp.float32),
            jax.ShapeDtypeStruct((_NROW, _NQP), jnp.float32),
            jax.ShapeDtypeStruct((_NROW, _NQP), jnp.float32),
            jax.ShapeDtypeStruct((_NQP,), jnp.float32),
        ),
        mesh=plsc.VectorSubcoreMesh(core_axis_name="c", subcore_axis_name="s",
                                    num_cores=2, num_subcores=16),
        compiler_params=pltpu.CompilerParams(needs_layout_passes=False),
        scratch_types=[
            pltpu.VMEM((12, _CH), jnp.float32),
            pltpu.VMEM((72 * 16,), jnp.float32),
            pltpu.VMEM((_NROW, _CH), jnp.float32),
            pltpu.VMEM((_NROW, _CH), jnp.float32),
            pltpu.VMEM((_NROW, _CH), jnp.float32),
            pltpu.VMEM((_CH,), jnp.float32),
            pltpu.SemaphoreType.DMA,
        ],
    )


@jax.jit
def kernel(reference_points, ego2img):
    rp = reference_points.astype(jnp.float32)[0]
    xyz = jnp.transpose(rp, (2, 0, 1)).reshape(12, _NQ)
    xyz = jnp.pad(xyz, ((0, 0), (0, _NQP - _NQ))).reshape(-1)

    m = ego2img.astype(jnp.float32)[0, :, :3, :]
    bits = m.view(jnp.int32)
    bits = (bits + jnp.int32(0x7FFF) + (jnp.right_shift(bits, 16) & 1))
    bits = bits & jnp.int32(-65536)
    mq = bits.view(jnp.float32)
    mats = jnp.broadcast_to(mq.reshape(72, 1), (72, 16)).reshape(-1)

    u_p, v_p, mask_f, cnt = _get_sc_call()(xyz, mats)

    u4 = u_p[:, :_NQ].reshape(_NP, 1, _NC, _NQ, 1)
    v4 = v_p[:, :_NQ].reshape(_NP, 1, _NC, _NQ, 1)
    parity = lax.broadcasted_iota(jnp.int32, (_NP, 1, _NC, _NQ, 2), 4)
    rpc = jnp.where(parity == 0, u4, v4)
    bev_mask = mask_f[:, :_NQ].reshape(_NP, 1, _NC, _NQ, 1).astype(bool)
    counts = cnt[:_NQ].reshape(1, _NQ)
    return rpc, bev_mask, counts
